# baseline (speedup 1.0000x reference)
import jax
import jax.numpy as jnp
from jax.experimental import pallas as pl

_Q = 1024
_DH = 128
_C = 8
_K = 75
_N = 100000
_NC = 1000
_NKB = 2048
_NPAD = 100352


def _mlp_kernel(x_ref, w1, b1, w2, b2, w3, b3, w4, b4, x1o, x2o, x3o, x4o):
    x = x_ref[...]
    h1 = jnp.maximum(
        jnp.dot(x, w1[...], preferred_element_type=jnp.float32) + b1[...], 0.0)
    x1o[...] = h1
    h2 = jnp.maximum(
        jnp.dot(h1, w2[...], preferred_element_type=jnp.float32) + b2[...], 0.0)
    x2o[...] = h2
    h3 = jnp.maximum(
        jnp.dot(h2, w3[...], preferred_element_type=jnp.float32) + b3[...], 0.0)
    x3o[...] = h3
    logits = jnp.dot(h3, w4[...], preferred_element_type=jnp.float32) + b4[...]
    m = jnp.max(logits, axis=-1, keepdims=True)
    e = jnp.exp(logits - m)
    x4o[...] = e / jnp.sum(e, axis=-1, keepdims=True)


def _mlp_forward(x, W1, b1, W2, b2, W3, b3, W4, b4):
    outs = [
        jax.ShapeDtypeStruct((_Q, _DH), jnp.float32),
        jax.ShapeDtypeStruct((_Q, _DH), jnp.float32),
        jax.ShapeDtypeStruct((_Q, _DH), jnp.float32),
        jax.ShapeDtypeStruct((_Q, _C), jnp.float32),
    ]
    return pl.pallas_call(_mlp_kernel, out_shape=outs)(
        x, W1, b1.reshape(1, -1), W2, b2.reshape(1, -1),
        W3, b3.reshape(1, -1), W4, b4.reshape(1, -1))


def _d2_kernel(q_ref, k_ref, o_ref):
    q = q_ref[...]
    k = k_ref[...]
    qq = jnp.sum(q * q, axis=1, keepdims=True)
    kk = jnp.sum(k * k, axis=1)
    qk = jax.lax.dot_general(q, k, (((1,), (1,)), ((), ())),
                             preferred_element_type=jnp.float32)
    d2 = qq + kk[None, :] - 2.0 * qk
    col = (pl.program_id(0) * _NKB
           + jax.lax.broadcasted_iota(jnp.int32, (1, _NKB), 1))
    o_ref[...] = jnp.where(col < _N, d2, jnp.inf)


def _d2(q, keys):
    d = q.shape[1]
    keys_p = jnp.pad(keys, ((0, _NPAD - _N), (0, 0)))
    grid = _NPAD // _NKB
    return pl.pallas_call(
        _d2_kernel,
        grid=(grid,),
        in_specs=[
            pl.BlockSpec((_Q, d), lambda i: (0, 0)),
            pl.BlockSpec((_NKB, d), lambda i: (i, 0)),
        ],
        out_specs=pl.BlockSpec((_Q, _NKB), lambda i: (0, i)),
        out_shape=jax.ShapeDtypeStruct((_Q, _NPAD), jnp.float32),
    )(q, keys_p)


def _epi_kernel(cali_ref, ls_ref,
                d0, l0, d1, l1, d2r, l2, d3, l3, d4, l4, o_ref):
    alpha = jnp.zeros((_Q, _C), dtype=jnp.float32)
    for d_ref, l_ref in ((d0, l0), (d1, l1), (d2r, l2), (d3, l3), (d4, l4)):
        d2v = jnp.maximum(d_ref[...], 0.0)
        safe = jnp.where(d2v > 0, d2v, 1.0)
        inv = jnp.where(d2v > 0, jax.lax.rsqrt(safe), 0.0)
        lab = l_ref[...]
        cols = []
        for c in range(_C):
            mism = (lab != ls_ref[0, c]).astype(jnp.float32)
            cols.append(jnp.sum(inv * mism, axis=1, keepdims=True))
        alpha = alpha + jnp.concatenate(cols, axis=1)
    cali = cali_ref[...]
    cnt_cols = []
    for c in range(_C):
        ge = (cali >= alpha[:, c:c + 1]).astype(jnp.float32)
        cnt_cols.append(jnp.sum(ge, axis=1, keepdims=True))
    o_ref[...] = jnp.concatenate(cnt_cols, axis=1) * (1.0 / _NC)


def kernel(input_tensor, W1, b1, W2, b2, W3, b3, W4, b4,
           keys0, keys1, keys2, keys3, keys4,
           cali_nonconformity, train_label, label_sample):
    x1, x2, x3, x4 = _mlp_forward(input_tensor, W1, b1, W2, b2, W3, b3, W4, b4)
    reps = (input_tensor, x1, x2, x3, x4)
    keys = (keys0, keys1, keys2, keys3, keys4)
    tl = train_label.astype(jnp.int32)
    epi_args = []
    for q, ks in zip(reps, keys):
        d2full = _d2(q, ks)
        neg_top, idx = jax.lax.top_k(-d2full, _K)
        epi_args.append(-neg_top)
        epi_args.append(tl[idx])
    cali = jnp.full((1, 1024), -jnp.inf, dtype=jnp.float32)
    cali = cali.at[0, :_NC].set(cali_nonconformity.astype(jnp.float32))
    ls = label_sample.astype(jnp.int32).reshape(1, _C)
    return pl.pallas_call(
        _epi_kernel,
        out_shape=jax.ShapeDtypeStruct((_Q, _C), jnp.float32),
    )(cali, ls, *epi_args)


# speedup vs baseline: 1.0014x; 1.0014x over previous
import jax
import jax.numpy as jnp
from jax.experimental import pallas as pl

_Q = 1024
_DH = 128
_C = 8
_K = 75
_N = 100000
_NC = 1000
_NKB = 2048
_NPAD = 100352


def _mlp_kernel(x_ref, w1, b1, w2, b2, w3, b3, w4, b4, x1o, x2o, x3o, x4o):
    x = x_ref[...]
    h1 = jnp.maximum(
        jnp.dot(x, w1[...], preferred_element_type=jnp.float32) + b1[...], 0.0)
    x1o[...] = h1
    h2 = jnp.maximum(
        jnp.dot(h1, w2[...], preferred_element_type=jnp.float32) + b2[...], 0.0)
    x2o[...] = h2
    h3 = jnp.maximum(
        jnp.dot(h2, w3[...], preferred_element_type=jnp.float32) + b3[...], 0.0)
    x3o[...] = h3
    logits = jnp.dot(h3, w4[...], preferred_element_type=jnp.float32) + b4[...]
    m = jnp.max(logits, axis=-1, keepdims=True)
    e = jnp.exp(logits - m)
    x4o[...] = e / jnp.sum(e, axis=-1, keepdims=True)


def _mlp_forward(x, W1, b1, W2, b2, W3, b3, W4, b4):
    outs = [
        jax.ShapeDtypeStruct((_Q, _DH), jnp.float32),
        jax.ShapeDtypeStruct((_Q, _DH), jnp.float32),
        jax.ShapeDtypeStruct((_Q, _DH), jnp.float32),
        jax.ShapeDtypeStruct((_Q, _C), jnp.float32),
    ]
    return pl.pallas_call(_mlp_kernel, out_shape=outs)(
        x, W1, b1.reshape(1, -1), W2, b2.reshape(1, -1),
        W3, b3.reshape(1, -1), W4, b4.reshape(1, -1))


def _d2_kernel(q_ref, k_ref, o_ref):
    q = q_ref[...]
    k = k_ref[...]
    qq = jnp.sum(q * q, axis=1, keepdims=True)
    kk = jnp.sum(k * k, axis=1)
    qk = jax.lax.dot_general(q, k, (((1,), (1,)), ((), ())),
                             preferred_element_type=jnp.float32)
    nd2 = 2.0 * qk - (qq + kk[None, :])
    col = (pl.program_id(0) * _NKB
           + jax.lax.broadcasted_iota(jnp.int32, (1, _NKB), 1))
    o_ref[...] = jnp.where(col < _N, nd2, -jnp.inf)


def _d2(q, keys):
    d = q.shape[1]
    keys_p = jnp.pad(keys, ((0, _NPAD - _N), (0, 0)))
    grid = _NPAD // _NKB
    return pl.pallas_call(
        _d2_kernel,
        grid=(grid,),
        in_specs=[
            pl.BlockSpec((_Q, d), lambda i: (0, 0)),
            pl.BlockSpec((_NKB, d), lambda i: (i, 0)),
        ],
        out_specs=pl.BlockSpec((_Q, _NKB), lambda i: (0, i)),
        out_shape=jax.ShapeDtypeStruct((_Q, _NPAD), jnp.float32),
    )(q, keys_p)


def _epi_kernel(cali_ref, ls_ref,
                d0, l0, d1, l1, d2r, l2, d3, l3, d4, l4, o_ref):
    alpha = jnp.zeros((_Q, _C), dtype=jnp.float32)
    for d_ref, l_ref in ((d0, l0), (d1, l1), (d2r, l2), (d3, l3), (d4, l4)):
        d2v = jnp.maximum(-d_ref[...], 0.0)
        safe = jnp.where(d2v > 0, d2v, 1.0)
        inv = jnp.where(d2v > 0, jax.lax.rsqrt(safe), 0.0)
        lab = l_ref[...]
        cols = []
        for c in range(_C):
            mism = (lab != ls_ref[0, c]).astype(jnp.float32)
            cols.append(jnp.sum(inv * mism, axis=1, keepdims=True))
        alpha = alpha + jnp.concatenate(cols, axis=1)
    cali = cali_ref[...]
    cnt_cols = []
    for c in range(_C):
        ge = (cali >= alpha[:, c:c + 1]).astype(jnp.float32)
        cnt_cols.append(jnp.sum(ge, axis=1, keepdims=True))
    o_ref[...] = jnp.concatenate(cnt_cols, axis=1) * (1.0 / _NC)


def kernel(input_tensor, W1, b1, W2, b2, W3, b3, W4, b4,
           keys0, keys1, keys2, keys3, keys4,
           cali_nonconformity, train_label, label_sample):
    x1, x2, x3, x4 = _mlp_forward(input_tensor, W1, b1, W2, b2, W3, b3, W4, b4)
    reps = (input_tensor, x1, x2, x3, x4)
    keys = (keys0, keys1, keys2, keys3, keys4)
    tl = train_label.astype(jnp.int32)
    epi_args = []
    for q, ks in zip(reps, keys):
        negd2 = _d2(q, ks)
        neg_top, idx = jax.lax.top_k(negd2, _K)
        epi_args.append(neg_top)
        epi_args.append(tl[idx])
    cali = jnp.full((1, 1024), -jnp.inf, dtype=jnp.float32)
    cali = cali.at[0, :_NC].set(cali_nonconformity.astype(jnp.float32))
    ls = label_sample.astype(jnp.int32).reshape(1, _C)
    return pl.pallas_call(
        _epi_kernel,
        out_shape=jax.ShapeDtypeStruct((_Q, _C), jnp.float32),
    )(cali, ls, *epi_args)
